# baseline (speedup 1.0000x reference)
import jax
import jax.numpy as jnp
from jax import lax
from jax.experimental import pallas as pl
from jax.experimental.pallas import tpu as pltpu
from jax.experimental.pallas import tpu_sc as plsc

NC, NS = 2, 16
NW = NC * NS
CB = 128
NJ = 200
NG = 3
NO = 2
BATCH = 4096
HIST = 200
DIM = 64
L = 16


def _body(table2_hbm, xt_hbm, out_hbm, idx_v, pidx_v, rows_v, outb_v, gsems, wsems):
    wid = lax.axis_index("s") * NC + lax.axis_index("c")
    b0 = wid * CB

    pltpu.sync_copy(xt_hbm.at[:, pl.ds(b0, CB)], idx_v)

    iotas = [lax.iota(jnp.int32, L) + (L * g) for g in range(8)]

    def start_gather(h, s):
        for g in range(8):
            pidx_v[s, pl.ds(L * g, L)] = lax.shift_right_logical(
                idx_v[h, pl.ds(L * g, L)], 1
            )
        pltpu.async_copy(
            table2_hbm.at[pidx_v.at[s]], rows_v.at[s], gsems.at[s]
        )

    def wait_gather(s):
        pltpu.make_async_copy(
            table2_hbm.at[pidx_v.at[s]], rows_v.at[s], gsems.at[s]
        ).wait()

    def start_write(h, o):
        pltpu.async_copy(
            outb_v.at[o], out_hbm.at[h, :, pl.ds(b0, CB)], wsems.at[o]
        )

    def wait_write(o):
        pltpu.make_async_copy(
            outb_v.at[o], out_hbm.at[0, :, pl.ds(b0, CB)], wsems.at[o]
        ).wait()

    def transpose_select(h, s, o):
        rows = rows_v.at[s]
        outb = outb_v.at[o]
        hvs = [
            lax.shift_left(
                lax.bitwise_and(idx_v[h, pl.ds(L * g, L)], 1), 6
            )
            for g in range(8)
        ]

        @pl.loop(0, DIM)
        def _(d):
            for g in range(8):
                val = plsc.load_gather(rows, [iotas[g], hvs[g] + d])
                outb[d, pl.ds(L * g, L)] = val

    def step(h, first, mid, last):
        s = h % NG
        if not last:
            start_gather(h + NG - 1, (h - 1) % NG)
        wait_gather(s)
        o = h % NO
        if not first:
            wait_write(o)
        transpose_select(h, s, o)
        start_write(h, o)

    for s in range(NG - 1):
        start_gather(s, s)

    step(0, first=True, mid=False, last=False)
    step(1, first=True, mid=False, last=False)

    @pl.loop(2, NJ - NG + 1)
    def _(h):
        step(h, first=False, mid=True, last=False)

    for h in range(NJ - NG + 1, NJ):
        step(h, first=False, mid=False, last=True)

    for o in range(NO):
        wait_write(o)


_gather = pl.kernel(
    _body,
    out_type=jax.ShapeDtypeStruct((HIST, DIM, BATCH), jnp.float32),
    mesh=plsc.VectorSubcoreMesh(
        core_axis_name="c", subcore_axis_name="s", num_cores=NC, num_subcores=NS
    ),
    scratch_types=[
        pltpu.VMEM((NJ, CB), jnp.int32),
        pltpu.VMEM((NG, CB), jnp.int32),
        pltpu.VMEM((NG, CB, 128), jnp.float32),
        pltpu.VMEM((NO, DIM, CB), jnp.float32),
        pltpu.SemaphoreType.DMA((NG,)),
        pltpu.SemaphoreType.DMA((NO,)),
    ],
    compiler_params=pltpu.CompilerParams(needs_layout_passes=False),
)


def kernel(x, table):
    table2 = table.reshape(500000, 128)
    xt = x.T
    outt = _gather(table2, xt)
    return outt.transpose(2, 0, 1)


# speedup vs baseline: 1.4580x; 1.4580x over previous
import jax
import jax.numpy as jnp
from jax import lax
from jax.experimental import pallas as pl
from jax.experimental.pallas import tpu as pltpu
from jax.experimental.pallas import tpu_sc as plsc

NC, NS = 2, 16
NW = NC * NS
CB = 128
NJ = 200
NG = 3
NO = 2
BATCH = 4096
HIST = 200
DIM = 64
L = 16


def _body(table2_hbm, xt_hbm, out_hbm, idx_v, pidx_v, rows_v, outb_v, gsems, wsems):
    wid = lax.axis_index("s") * NC + lax.axis_index("c")
    b0 = wid * CB

    pltpu.sync_copy(xt_hbm.at[:, pl.ds(b0, CB)], idx_v)

    iotas = [lax.iota(jnp.int32, L) + (L * g) for g in range(8)]

    def start_gather(h, s):
        for g in range(8):
            pidx_v[s, pl.ds(L * g, L)] = lax.shift_right_logical(
                idx_v[h, pl.ds(L * g, L)], 1
            )
        pltpu.async_copy(
            table2_hbm.at[pidx_v.at[s]], rows_v.at[s], gsems.at[s]
        )

    def wait_gather(s):
        pltpu.make_async_copy(
            table2_hbm.at[pidx_v.at[s]], rows_v.at[s], gsems.at[s]
        ).wait()

    def start_write(h, o):
        pltpu.async_copy(
            outb_v.at[o], out_hbm.at[h, :, pl.ds(b0, CB)], wsems.at[o]
        )

    def wait_write(o):
        pltpu.make_async_copy(
            outb_v.at[o], out_hbm.at[0, :, pl.ds(b0, CB)], wsems.at[o]
        ).wait()

    def transpose_select(h, s, o):
        rows = rows_v.at[s]
        outb = outb_v.at[o]
        hvs = [
            lax.shift_left(
                lax.bitwise_and(idx_v[h, pl.ds(L * g, L)], 1), 6
            )
            for g in range(8)
        ]

        @plsc.parallel_loop(0, DIM, unroll=8)
        def _(d):
            for g in range(8):
                val = plsc.load_gather(rows, [iotas[g], hvs[g] + d])
                outb[d, pl.ds(L * g, L)] = val

    def step(h, first, mid, last):
        s = h % NG
        if not last:
            start_gather(h + NG - 1, (h - 1) % NG)
        wait_gather(s)
        o = h % NO
        if not first:
            wait_write(o)
        transpose_select(h, s, o)
        start_write(h, o)

    for s in range(NG - 1):
        start_gather(s, s)

    step(0, first=True, mid=False, last=False)
    step(1, first=True, mid=False, last=False)

    @pl.loop(2, NJ - NG + 1)
    def _(h):
        step(h, first=False, mid=True, last=False)

    for h in range(NJ - NG + 1, NJ):
        step(h, first=False, mid=False, last=True)

    for o in range(NO):
        wait_write(o)


_gather = pl.kernel(
    _body,
    out_type=jax.ShapeDtypeStruct((HIST, DIM, BATCH), jnp.float32),
    mesh=plsc.VectorSubcoreMesh(
        core_axis_name="c", subcore_axis_name="s", num_cores=NC, num_subcores=NS
    ),
    scratch_types=[
        pltpu.VMEM((NJ, CB), jnp.int32),
        pltpu.VMEM((NG, CB), jnp.int32),
        pltpu.VMEM((NG, CB, 128), jnp.float32),
        pltpu.VMEM((NO, DIM, CB), jnp.float32),
        pltpu.SemaphoreType.DMA((NG,)),
        pltpu.SemaphoreType.DMA((NO,)),
    ],
    compiler_params=pltpu.CompilerParams(needs_layout_passes=False),
)


def kernel(x, table):
    table2 = table.reshape(500000, 128)
    xt = x.T
    outt = _gather(table2, xt)
    return outt.transpose(2, 0, 1)
